# baseline (speedup 1.0000x reference)
import functools

import jax
import jax.numpy as jnp
from jax import lax
from jax.experimental import pallas as pl
from jax.experimental.pallas import tpu as pltpu
from jax.experimental.pallas import tpu_sc as plsc


def _sc_gather(ids_t, table, *, num_workers):
    n_rows, n_tok = ids_t.shape
    dim = table.shape[1]
    chunk = n_tok // num_workers
    sub = chunk // 128
    mesh = plsc.VectorSubcoreMesh(core_axis_name="c", subcore_axis_name="s")

    @functools.partial(
        pl.kernel,
        out_type=jax.ShapeDtypeStruct(
            (n_rows, num_workers, 128, sub * dim), jnp.float32
        ),
        mesh=mesh,
        scratch_types=[
            pltpu.VMEM((n_rows, chunk), jnp.int32),
            [
                [pltpu.VMEM((128, dim), jnp.float32) for _ in range(4)]
                for _ in range(2)
            ],
            [pltpu.SemaphoreType.DMA for _ in range(2)],
            [pltpu.SemaphoreType.DMA for _ in range(2)],
        ],
        compiler_params=pltpu.CompilerParams(use_tc_tiling_on_sc=False),
    )
    def k(ids_hbm, table_hbm, out_hbm, idx_v, bufs, g_sems, w_sems):
        wid = lax.axis_index("s") * 2 + lax.axis_index("c")
        base = wid * chunk
        pltpu.sync_copy(ids_hbm.at[:, pl.ds(base, chunk)], idx_v)

        def start_gathers(j, b):
            descs = []
            for p in range(sub):
                descs.append(
                    pltpu.async_copy(
                        table_hbm.at[idx_v.at[j, pl.ds(p * 128, 128)]],
                        bufs[b][p],
                        g_sems[b],
                    )
                )
            return descs

        def write_out(j, b):
            descs = []
            for p in range(sub):
                descs.append(
                    pltpu.async_copy(
                        bufs[b][p],
                        out_hbm.at[j, wid, :, pl.ds(p * dim, dim)],
                        w_sems[b],
                    )
                )
            return descs

        def body(g, _):
            j0 = g * 2
            j1 = j0 + 1
            gd0 = start_gathers(j0, 0)
            gd1 = start_gathers(j1, 1)
            for d in gd0:
                d.wait()
            wd0 = write_out(j0, 0)
            for d in gd1:
                d.wait()
            wd1 = write_out(j1, 1)
            for d in wd0 + wd1:
                d.wait()
            return 0

        lax.fori_loop(0, n_rows // 2, body, 0)

    return k(ids_t, table)


def _tc_transpose(mid):
    n_rows, num_workers, r128, sd = mid.shape
    dim = 32
    sub = sd // dim
    chunk = sub * r128
    n_tok = num_workers * chunk

    wblk = 4

    def body(x_ref, y_ref):
        ident = jax.lax.broadcasted_iota(jnp.int32, (r128, r128), 0)
        ident = (ident == jax.lax.broadcasted_iota(jnp.int32, (r128, r128), 1))
        ident = ident.astype(jnp.float32)
        for u in range(wblk):
            x = x_ref[0, u]
            xt = jax.lax.dot_general(
                x, ident,
                dimension_numbers=(((0,), (0,)), ((), ())),
                precision=jax.lax.Precision.HIGHEST,
                preferred_element_type=jnp.float32,
            )
            y_ref[0, :, pl.ds(u * chunk, chunk)] = jnp.concatenate(
                [xt[p * dim:(p + 1) * dim, :] for p in range(sub)], axis=1
            )

    return pl.pallas_call(
        body,
        grid=(n_rows, num_workers // wblk),
        in_specs=[
            pl.BlockSpec((1, wblk, r128, sd), lambda j, w: (j, w, 0, 0)),
        ],
        out_specs=pl.BlockSpec(
            (1, dim, wblk * chunk), lambda j, w: (j, 0, w)
        ),
        out_shape=jax.ShapeDtypeStruct((n_rows, dim, n_tok), jnp.float32),
    )(mid)


def kernel(token_ids, table):
    ids_t = token_ids.T.astype(jnp.int32)
    mid = _sc_gather(ids_t, table, num_workers=32)
    out_t = _tc_transpose(mid)
    return out_t.transpose(2, 0, 1)


# speedup vs baseline: 1.2026x; 1.2026x over previous
import functools

import jax
import jax.numpy as jnp
from jax import lax
from jax.experimental import pallas as pl
from jax.experimental.pallas import tpu as pltpu
from jax.experimental.pallas import tpu_sc as plsc


def _sc_gather(ids_t, table, *, num_workers):
    n_rows, n_tok = ids_t.shape
    dim = table.shape[1]
    chunk = n_tok // num_workers
    sub = chunk // 128
    mesh = plsc.VectorSubcoreMesh(core_axis_name="c", subcore_axis_name="s")

    @functools.partial(
        pl.kernel,
        out_type=jax.ShapeDtypeStruct(
            (n_rows, num_workers, 128, sub * dim), jnp.float32
        ),
        mesh=mesh,
        scratch_types=[
            pltpu.VMEM((n_rows, chunk), jnp.int32),
            [
                [pltpu.VMEM((128, dim), jnp.float32) for _ in range(4)]
                for _ in range(2)
            ],
            [pltpu.SemaphoreType.DMA for _ in range(2)],
            [pltpu.SemaphoreType.DMA for _ in range(2)],
        ],
        compiler_params=pltpu.CompilerParams(use_tc_tiling_on_sc=False),
    )
    def k(ids_hbm, table_hbm, out_hbm, idx_v, bufs, g_sems, w_sems):
        wid = lax.axis_index("s") * 2 + lax.axis_index("c")
        base = wid * chunk
        pltpu.sync_copy(ids_hbm.at[:, pl.ds(base, chunk)], idx_v)

        def start_gathers(j, b):
            descs = []
            for p in range(sub):
                descs.append(
                    pltpu.async_copy(
                        table_hbm.at[idx_v.at[j, pl.ds(p * 128, 128)]],
                        bufs[b][p],
                        g_sems[b],
                    )
                )
            return descs

        def write_out(j, b):
            descs = []
            for p in range(sub):
                descs.append(
                    pltpu.async_copy(
                        bufs[b][p],
                        out_hbm.at[j, wid, :, pl.ds(p * dim, dim)],
                        w_sems[b],
                    )
                )
            return descs

        def body(g, _):
            j0 = g * 2
            j1 = j0 + 1
            gd0 = start_gathers(j0, 0)
            gd1 = start_gathers(j1, 1)
            for d in gd0:
                d.wait()
            wd0 = write_out(j0, 0)
            for d in gd1:
                d.wait()
            wd1 = write_out(j1, 1)
            for d in wd0 + wd1:
                d.wait()
            return 0

        lax.fori_loop(0, n_rows // 2, body, 0)

    return k(ids_t, table)


def _tc_transpose(mid):
    n_rows, num_workers, r128, sd = mid.shape
    dim = 32
    sub = sd // dim
    chunk = sub * r128
    n_tok = num_workers * chunk

    wblk = 8

    def body(x_ref, y_ref):
        for u in range(wblk):
            xt = x_ref[0, u].T
            for p in range(sub):
                y_ref[0, :, pl.ds(u * chunk + p * 128, 128)] = (
                    xt[p * dim:(p + 1) * dim, :]
                )

    return pl.pallas_call(
        body,
        grid=(n_rows, num_workers // wblk),
        in_specs=[
            pl.BlockSpec((1, wblk, r128, sd), lambda j, w: (j, w, 0, 0)),
        ],
        out_specs=pl.BlockSpec(
            (1, dim, wblk * chunk), lambda j, w: (j, 0, w)
        ),
        out_shape=jax.ShapeDtypeStruct((n_rows, dim, n_tok), jnp.float32),
    )(mid)


def kernel(token_ids, table):
    ids_t = token_ids.T.astype(jnp.int32)
    mid = _sc_gather(ids_t, table, num_workers=32)
    out_t = _tc_transpose(mid)
    return out_t.transpose(2, 0, 1)


# speedup vs baseline: 1.2765x; 1.0614x over previous
import functools

import jax
import jax.numpy as jnp
from jax import lax
from jax.experimental import pallas as pl
from jax.experimental.pallas import tpu as pltpu
from jax.experimental.pallas import tpu_sc as plsc


def _sc_gather(ids_t, table, *, num_workers):
    n_rows, n_tok = ids_t.shape
    dim = table.shape[1]
    chunk = n_tok // num_workers
    sub = chunk // 128
    mesh = plsc.VectorSubcoreMesh(core_axis_name="c", subcore_axis_name="s")

    @functools.partial(
        pl.kernel,
        out_type=jax.ShapeDtypeStruct(
            (n_rows, num_workers, 128, sub * dim), jnp.float32
        ),
        mesh=mesh,
        scratch_types=[
            pltpu.VMEM((n_rows, chunk), jnp.int32),
            [
                [pltpu.VMEM((128, dim), jnp.float32) for _ in range(4)]
                for _ in range(2)
            ],
            [pltpu.SemaphoreType.DMA for _ in range(2)],
            [pltpu.SemaphoreType.DMA for _ in range(2)],
        ],
        compiler_params=pltpu.CompilerParams(use_tc_tiling_on_sc=False),
    )
    def k(ids_hbm, table_hbm, out_hbm, idx_v, bufs, g_sems, w_sems):
        wid = lax.axis_index("s") * 2 + lax.axis_index("c")
        base = wid * chunk
        pltpu.sync_copy(ids_hbm.at[:, pl.ds(base, chunk)], idx_v)

        def start_gathers(j, b):
            descs = []
            for p in range(sub):
                descs.append(
                    pltpu.async_copy(
                        table_hbm.at[idx_v.at[j, pl.ds(p * 128, 128)]],
                        bufs[b][p],
                        g_sems[b],
                    )
                )
            return descs

        def write_out(j, b):
            descs = []
            for p in range(sub):
                descs.append(
                    pltpu.async_copy(
                        bufs[b][p],
                        out_hbm.at[j, wid, :, pl.ds(p * dim, dim)],
                        w_sems[b],
                    )
                )
            return descs

        def body(g, _):
            j0 = g * 2
            j1 = j0 + 1
            gd0 = start_gathers(j0, 0)
            gd1 = start_gathers(j1, 1)
            for d in gd0:
                d.wait()
            wd0 = write_out(j0, 0)
            for d in gd1:
                d.wait()
            wd1 = write_out(j1, 1)
            for d in wd0 + wd1:
                d.wait()
            return 0

        lax.fori_loop(0, n_rows // 2, body, 0)

    return k(ids_t, table)


def _tc_transpose(mid):
    n_rows, num_workers, r128, sd = mid.shape
    dim = 32
    sub = sd // dim
    chunk = sub * r128
    n_tok = num_workers * chunk

    wblk = 16

    def body(x_ref, y_ref):
        for u in range(wblk):
            xt = x_ref[0, u].T
            for p in range(sub):
                y_ref[0, :, pl.ds(u * chunk + p * 128, 128)] = (
                    xt[p * dim:(p + 1) * dim, :]
                )

    return pl.pallas_call(
        body,
        grid=(n_rows, num_workers // wblk),
        in_specs=[
            pl.BlockSpec((1, wblk, r128, sd), lambda j, w: (j, w, 0, 0)),
        ],
        out_specs=pl.BlockSpec(
            (1, dim, wblk * chunk), lambda j, w: (j, 0, w)
        ),
        out_shape=jax.ShapeDtypeStruct((n_rows, dim, n_tok), jnp.float32),
    )(mid)


def kernel(token_ids, table):
    ids_t = token_ids.T.astype(jnp.int32)
    mid = _sc_gather(ids_t, table, num_workers=32)
    out_t = _tc_transpose(mid)
    return out_t.transpose(2, 0, 1)


# speedup vs baseline: 1.3401x; 1.0499x over previous
import functools

import jax
import jax.numpy as jnp
from jax import lax
from jax.experimental import pallas as pl
from jax.experimental.pallas import tpu as pltpu
from jax.experimental.pallas import tpu_sc as plsc


def _sc_gather(ids_t, table, *, num_workers):
    n_rows, n_tok = ids_t.shape
    dim = table.shape[1]
    chunk = n_tok // num_workers
    sub = chunk // 128
    mesh = plsc.VectorSubcoreMesh(core_axis_name="c", subcore_axis_name="s")

    @functools.partial(
        pl.kernel,
        out_type=jax.ShapeDtypeStruct(
            (n_rows, num_workers, 128, sub * dim), jnp.float32
        ),
        mesh=mesh,
        scratch_types=[
            pltpu.VMEM((n_rows, chunk), jnp.int32),
            [
                [pltpu.VMEM((128, dim), jnp.float32) for _ in range(4)]
                for _ in range(2)
            ],
            [pltpu.SemaphoreType.DMA for _ in range(2)],
            [pltpu.SemaphoreType.DMA for _ in range(2)],
        ],
        compiler_params=pltpu.CompilerParams(use_tc_tiling_on_sc=False),
    )
    def k(ids_hbm, table_hbm, out_hbm, idx_v, bufs, g_sems, w_sems):
        wid = lax.axis_index("s") * 2 + lax.axis_index("c")
        base = wid * chunk
        pltpu.sync_copy(ids_hbm.at[:, pl.ds(base, chunk)], idx_v)

        def start_gathers(j, b):
            descs = []
            for p in range(sub):
                descs.append(
                    pltpu.async_copy(
                        table_hbm.at[idx_v.at[j, pl.ds(p * 128, 128)]],
                        bufs[b][p],
                        g_sems[b],
                    )
                )
            return descs

        def write_out(j, b):
            descs = []
            for p in range(sub):
                descs.append(
                    pltpu.async_copy(
                        bufs[b][p],
                        out_hbm.at[j, wid, :, pl.ds(p * dim, dim)],
                        w_sems[b],
                    )
                )
            return descs

        def body(g, _):
            j0 = g * 2
            j1 = j0 + 1
            gd0 = start_gathers(j0, 0)
            gd1 = start_gathers(j1, 1)
            for d in gd0:
                d.wait()
            wd0 = write_out(j0, 0)
            for d in gd1:
                d.wait()
            wd1 = write_out(j1, 1)
            for d in wd0 + wd1:
                d.wait()
            return 0

        lax.fori_loop(0, n_rows // 2, body, 0)

    return k(ids_t, table)


def _tc_transpose(mid):
    n_rows, num_workers, r128, sd = mid.shape
    dim = 32
    sub = sd // dim
    chunk = sub * r128
    n_tok = num_workers * chunk

    wblk = 32

    def body(x_ref, y_ref):
        for u in range(wblk):
            xt = x_ref[0, u].T
            for p in range(sub):
                y_ref[0, :, pl.ds(u * chunk + p * 128, 128)] = (
                    xt[p * dim:(p + 1) * dim, :]
                )

    return pl.pallas_call(
        body,
        grid=(n_rows, num_workers // wblk),
        in_specs=[
            pl.BlockSpec((1, wblk, r128, sd), lambda j, w: (j, w, 0, 0)),
        ],
        out_specs=pl.BlockSpec(
            (1, dim, wblk * chunk), lambda j, w: (j, 0, w)
        ),
        out_shape=jax.ShapeDtypeStruct((n_rows, dim, n_tok), jnp.float32),
    )(mid)


def kernel(token_ids, table):
    ids_t = token_ids.T.astype(jnp.int32)
    mid = _sc_gather(ids_t, table, num_workers=32)
    out_t = _tc_transpose(mid)
    return out_t.transpose(2, 0, 1)
